# Initial kernel scaffold; baseline (speedup 1.0000x reference)
import jax
import jax.numpy as jnp
from jax.experimental import pallas as pl


def kernel(x, table):
    raise NotImplementedError("write your pallas kernel here")


# speedup vs baseline: 1.8446x; 1.8446x over previous
import functools

import jax
import jax.numpy as jnp
from jax import lax
from jax.experimental import pallas as pl
from jax.experimental.pallas import tpu as pltpu
from jax.experimental.pallas import tpu_sc as plsc

EMBED = 64
CHUNK = 128
CPG = 8
ROWS_PER_GROUP = CHUNK * CPG


@functools.lru_cache(maxsize=None)
def _build_gather(total_rows: int):
    info = plsc.get_sparse_core_info()
    nc, ns = info.num_cores, info.num_subcores
    nw = nc * ns
    assert total_rows % (nw * ROWS_PER_GROUP) == 0
    groups_per_worker = total_rows // (nw * ROWS_PER_GROUP)
    mesh = plsc.VectorSubcoreMesh(core_axis_name="c", subcore_axis_name="s")

    @functools.partial(
        pl.kernel,
        mesh=mesh,
        out_type=jax.ShapeDtypeStruct((total_rows, EMBED), jnp.float32),
        scratch_types=[
            pltpu.VMEM((CPG, CHUNK), jnp.int32),
            pltpu.VMEM((ROWS_PER_GROUP, EMBED), jnp.float32),
            pltpu.SemaphoreType.DMA,
        ],
        compiler_params=pltpu.CompilerParams(use_tc_tiling_on_sc=False),
    )
    def gather(idx_hbm, table_hbm, out_hbm, idx_v, rows_v, sem):
        wid = lax.axis_index("s") * nc + lax.axis_index("c")
        gbase = wid * groups_per_worker

        def group(g, carry):
            chunk0 = (gbase + g) * CPG
            pltpu.sync_copy(idx_hbm.at[pl.ds(chunk0, CPG)], idx_v)
            copies = [
                pltpu.async_copy(
                    table_hbm.at[idx_v.at[c]],
                    rows_v.at[pl.ds(c * CHUNK, CHUNK)],
                    sem,
                )
                for c in range(CPG)
            ]
            for cp in copies:
                cp.wait()
            row0 = (gbase + g) * ROWS_PER_GROUP
            pltpu.sync_copy(rows_v, out_hbm.at[pl.ds(row0, ROWS_PER_GROUP)])
            return carry

        lax.fori_loop(0, groups_per_worker, group, 0)

    return gather


def kernel(x, table):
    batch, hist = x.shape
    total = batch * hist
    idx = x.reshape(total // CHUNK, CHUNK).astype(jnp.int32)
    out = _build_gather(total)(idx, table)
    return out.reshape(batch, hist, EMBED)


# speedup vs baseline: 1.8743x; 1.0161x over previous
import functools

import jax
import jax.numpy as jnp
from jax import lax
from jax.experimental import pallas as pl
from jax.experimental.pallas import tpu as pltpu
from jax.experimental.pallas import tpu_sc as plsc

EMBED = 64
CHUNK = 128
NBUF = 8
LAG = 4


@functools.lru_cache(maxsize=None)
def _build_gather(total_rows: int):
    info = plsc.get_sparse_core_info()
    nc, ns = info.num_cores, info.num_subcores
    nw = nc * ns
    assert total_rows % (nw * CHUNK * NBUF) == 0
    groups = total_rows // (nw * CHUNK)
    bodies = groups // NBUF
    mesh = plsc.VectorSubcoreMesh(core_axis_name="c", subcore_axis_name="s")

    @functools.partial(
        pl.kernel,
        mesh=mesh,
        out_type=jax.ShapeDtypeStruct((total_rows, EMBED), jnp.float32),
        scratch_types=(
            [
                pltpu.VMEM((groups, CHUNK), jnp.int32),
                pltpu.VMEM((NBUF * CHUNK, EMBED), jnp.float32),
            ]
            + [pltpu.SemaphoreType.DMA] * (2 * NBUF)
        ),
        compiler_params=pltpu.CompilerParams(use_tc_tiling_on_sc=False),
    )
    def gather(idx_hbm, table_hbm, out_hbm, idx_v, rows_v, *sems):
        gsem = sems[:NBUF]
        osem = sems[NBUF:]
        wid = lax.axis_index("s") * nc + lax.axis_index("c")
        row_base = wid * (groups * CHUNK)

        pltpu.sync_copy(idx_hbm.at[pl.ds(wid * groups, groups)], idx_v)

        def drain_gather(sp):
            pltpu.make_async_copy(
                table_hbm.at[pl.ds(0, CHUNK)],
                rows_v.at[pl.ds(sp * CHUNK, CHUNK)],
                gsem[sp],
            ).wait()

        def fire_write(sp, gp):
            pltpu.async_copy(
                rows_v.at[pl.ds(sp * CHUNK, CHUNK)],
                out_hbm.at[pl.ds(row_base + gp * CHUNK, CHUNK)],
                osem[sp],
            )

        def drain_write(sp):
            pltpu.make_async_copy(
                rows_v.at[pl.ds(sp * CHUNK, CHUNK)],
                out_hbm.at[pl.ds(row_base, CHUNK)],
                osem[sp],
            ).wait()

        def body(t, carry):
            for b in range(NBUF):
                g = t * NBUF + b
                sp = (b - LAG) % NBUF

                @pl.when(g >= LAG)
                def _(sp=sp, g=g):
                    drain_gather(sp)
                    fire_write(sp, g - LAG)

                @pl.when(g >= NBUF)
                def _(b=b):
                    drain_write(b)

                pltpu.async_copy(
                    table_hbm.at[idx_v.at[g]],
                    rows_v.at[pl.ds(b * CHUNK, CHUNK)],
                    gsem[b],
                )
            return carry

        lax.fori_loop(0, bodies, body, 0)

        for k in range(LAG):
            gp = groups - LAG + k
            sp = gp % NBUF
            drain_gather(sp)
            fire_write(sp, gp)
        for b in range(NBUF):
            drain_write(b)

    return gather


def kernel(x, table):
    batch, hist = x.shape
    total = batch * hist
    idx = x.reshape(total // CHUNK, CHUNK).astype(jnp.int32)
    out = _build_gather(total)(idx, table)
    return out.reshape(batch, hist, EMBED)


# speedup vs baseline: 1.8749x; 1.0003x over previous
import functools

import jax
import jax.numpy as jnp
from jax import lax
from jax.experimental import pallas as pl
from jax.experimental.pallas import tpu as pltpu
from jax.experimental.pallas import tpu_sc as plsc

EMBED = 64
CHUNK = 128
NBUF = 8
LAG = 4


@functools.lru_cache(maxsize=None)
def _build_gather(total_rows: int):
    info = plsc.get_sparse_core_info()
    nc, ns = info.num_cores, info.num_subcores
    nw = nc * ns
    assert total_rows % (nw * CHUNK * NBUF) == 0
    groups = total_rows // (nw * CHUNK)
    bodies = groups // NBUF
    mesh = plsc.VectorSubcoreMesh(core_axis_name="c", subcore_axis_name="s")

    @functools.partial(
        pl.kernel,
        mesh=mesh,
        out_type=jax.ShapeDtypeStruct((total_rows, EMBED), jnp.float32),
        scratch_types=(
            [
                pltpu.VMEM((groups, CHUNK), jnp.int32),
                pltpu.VMEM((NBUF * CHUNK, EMBED), jnp.float32),
            ]
            + [pltpu.SemaphoreType.DMA] * (2 * NBUF)
        ),
        compiler_params=pltpu.CompilerParams(use_tc_tiling_on_sc=False),
    )
    def gather(idx_hbm, table_hbm, out_hbm, idx_v, rows_v, *sems):
        gsem = sems[:NBUF]
        osem = sems[NBUF:]
        wid = lax.axis_index("s") * nc + lax.axis_index("c")
        row_base = wid * (groups * CHUNK)

        pltpu.sync_copy(idx_hbm.at[pl.ds(wid * groups, groups)], idx_v)

        def drain_gather(sp):
            pltpu.make_async_copy(
                table_hbm.at[pl.ds(0, CHUNK)],
                rows_v.at[pl.ds(sp * CHUNK, CHUNK)],
                gsem[sp],
            ).wait()

        def fire_write(sp, gp):
            pltpu.async_copy(
                rows_v.at[pl.ds(sp * CHUNK, CHUNK)],
                out_hbm.at[pl.ds(row_base + gp * CHUNK, CHUNK)],
                osem[sp],
            )

        def drain_write(sp):
            pltpu.make_async_copy(
                rows_v.at[pl.ds(sp * CHUNK, CHUNK)],
                out_hbm.at[pl.ds(row_base, CHUNK)],
                osem[sp],
            ).wait()

        def body(t, carry):
            for b in range(NBUF):
                g = t * NBUF + b
                sp = (b - LAG) % NBUF

                @pl.when(g >= LAG)
                def _(sp=sp, g=g):
                    drain_gather(sp)
                    fire_write(sp, g - LAG)

                @pl.when(g >= NBUF)
                def _(b=b):
                    drain_write(b)

                pltpu.async_copy(
                    table_hbm.at[idx_v.at[g]],
                    rows_v.at[pl.ds(b * CHUNK, CHUNK)],
                    gsem[b],
                )
            return carry

        lax.fori_loop(0, bodies, body, 0)

        for k in range(LAG):
            gp = groups - LAG + k
            sp = gp % NBUF
            drain_gather(sp)
            fire_write(sp, gp)
        for b in range(NBUF):
            drain_write(b)

    return gather


def kernel(x, table):
    batch, hist = x.shape
    total = batch * hist
    idx = x.reshape(total // CHUNK, CHUNK).astype(jnp.int32)
    vocab = table.shape[0]
    table_lin = table.reshape(vocab * EMBED).reshape(vocab, EMBED)
    out = _build_gather(total)(idx, table_lin)
    return out.reshape(total * EMBED).reshape(batch, hist, EMBED)


# speedup vs baseline: 1.8881x; 1.0071x over previous
import functools

import jax
import jax.numpy as jnp
from jax import lax
from jax.experimental import pallas as pl
from jax.experimental.pallas import tpu as pltpu
from jax.experimental.pallas import tpu_sc as plsc

EMBED = 64
CHUNK = 128
NBUF = 10
LAG = 8


@functools.lru_cache(maxsize=None)
def _build_gather(total_rows: int):
    info = plsc.get_sparse_core_info()
    nc, ns = info.num_cores, info.num_subcores
    nw = nc * ns
    assert total_rows % (nw * CHUNK * NBUF) == 0
    groups = total_rows // (nw * CHUNK)
    bodies = groups // NBUF
    mesh = plsc.VectorSubcoreMesh(core_axis_name="c", subcore_axis_name="s")

    @functools.partial(
        pl.kernel,
        mesh=mesh,
        out_type=jax.ShapeDtypeStruct((total_rows, EMBED), jnp.float32),
        scratch_types=(
            [
                pltpu.VMEM((groups, CHUNK), jnp.int32),
                pltpu.VMEM((NBUF * CHUNK, EMBED), jnp.float32),
            ]
            + [pltpu.SemaphoreType.DMA] * (2 * NBUF)
        ),
        compiler_params=pltpu.CompilerParams(use_tc_tiling_on_sc=False),
    )
    def gather(idx_hbm, table_hbm, out_hbm, idx_v, rows_v, *sems):
        gsem = sems[:NBUF]
        osem = sems[NBUF:]
        wid = lax.axis_index("s") * nc + lax.axis_index("c")
        row_base = wid * (groups * CHUNK)

        pltpu.sync_copy(idx_hbm.at[pl.ds(wid * groups, groups)], idx_v)

        def drain_gather(sp):
            pltpu.make_async_copy(
                table_hbm.at[pl.ds(0, CHUNK)],
                rows_v.at[pl.ds(sp * CHUNK, CHUNK)],
                gsem[sp],
            ).wait()

        def fire_write(sp, gp):
            pltpu.async_copy(
                rows_v.at[pl.ds(sp * CHUNK, CHUNK)],
                out_hbm.at[pl.ds(row_base + gp * CHUNK, CHUNK)],
                osem[sp],
            )

        def drain_write(sp):
            pltpu.make_async_copy(
                rows_v.at[pl.ds(sp * CHUNK, CHUNK)],
                out_hbm.at[pl.ds(row_base, CHUNK)],
                osem[sp],
            ).wait()

        def body(t, carry):
            for b in range(NBUF):
                g = t * NBUF + b
                sp = (b - LAG) % NBUF

                @pl.when(g >= NBUF)
                def _(b=b):
                    drain_write(b)

                pltpu.async_copy(
                    table_hbm.at[idx_v.at[g]],
                    rows_v.at[pl.ds(b * CHUNK, CHUNK)],
                    gsem[b],
                )

                @pl.when(g >= LAG)
                def _(sp=sp, g=g):
                    drain_gather(sp)
                    fire_write(sp, g - LAG)
            return carry

        lax.fori_loop(0, bodies, body, 0)

        for k in range(LAG):
            gp = groups - LAG + k
            sp = gp % NBUF
            drain_gather(sp)
            fire_write(sp, gp)
        for b in range(NBUF):
            drain_write(b)

    return gather


def kernel(x, table):
    batch, hist = x.shape
    total = batch * hist
    idx = x.reshape(total // CHUNK, CHUNK).astype(jnp.int32)
    vocab = table.shape[0]
    table_lin = table.reshape(vocab * EMBED).reshape(vocab, EMBED)
    out = _build_gather(total)(idx, table_lin)
    return out.reshape(total * EMBED).reshape(batch, hist, EMBED)
